# baseline (device time: 13612 ns/iter reference)
import jax
import jax.numpy as jnp
from jax import lax
from jax.experimental import pallas as pl
from jax.experimental.pallas import tpu as pltpu

N_DEV = 32
Z_GROUP = 4
P_GROUP = 8


def kernel(x):
    m, n = x.shape

    def body(x_ref, out_ref, g1_ref, g2_ref, send1, recv1, send2, recv2):
        my = lax.axis_index("i")

        barrier_sem = pltpu.get_barrier_semaphore()
        for k in range(1, N_DEV):
            peer = lax.rem(my + k, N_DEV)
            pl.semaphore_signal(
                barrier_sem, inc=1,
                device_id=(peer,),
                device_id_type=pl.DeviceIdType.MESH,
            )

        g1_ref[0, 0, :] = jnp.sum(x_ref[:, :], axis=0)

        pl.semaphore_wait(barrier_sem, N_DEV - 1)

        rdmas1 = []
        for j in range(1, Z_GROUP):
            peer = lax.rem(my + 8 * j, N_DEV)
            rdma = pltpu.make_async_remote_copy(
                src_ref=g1_ref.at[0],
                dst_ref=g1_ref.at[j],
                send_sem=send1.at[j],
                recv_sem=recv1.at[j],
                device_id=(peer,),
                device_id_type=pl.DeviceIdType.MESH,
            )
            rdma.start()
            rdmas1.append(rdma)
        for rdma in rdmas1:
            rdma.wait()

        g2_ref[0, 0, :] = jnp.sum(g1_ref[:, 0, :], axis=0)

        q = lax.rem(my, P_GROUP)
        base = my - q
        rdmas2 = []
        for j in range(1, P_GROUP):
            peer = base + lax.rem(q + j, P_GROUP)
            rdma = pltpu.make_async_remote_copy(
                src_ref=g2_ref.at[0],
                dst_ref=g2_ref.at[j],
                send_sem=send2.at[j],
                recv_sem=recv2.at[j],
                device_id=(peer,),
                device_id_type=pl.DeviceIdType.MESH,
            )
            rdma.start()
            rdmas2.append(rdma)
        for rdma in rdmas2:
            rdma.wait()

        out_ref[0, :] = jnp.sum(g2_ref[:, 0, :], axis=0)

    return pl.pallas_call(
        body,
        out_shape=jax.ShapeDtypeStruct((1, n), jnp.float32),
        in_specs=[pl.BlockSpec(memory_space=pltpu.VMEM)],
        out_specs=pl.BlockSpec(memory_space=pltpu.VMEM),
        scratch_shapes=[
            pltpu.VMEM((Z_GROUP, 1, n), jnp.float32),
            pltpu.VMEM((P_GROUP, 1, n), jnp.float32),
            pltpu.SemaphoreType.DMA((Z_GROUP,)),
            pltpu.SemaphoreType.DMA((Z_GROUP,)),
            pltpu.SemaphoreType.DMA((P_GROUP,)),
            pltpu.SemaphoreType.DMA((P_GROUP,)),
        ],
        compiler_params=pltpu.CompilerParams(collective_id=0),
    )(x)
